# baseline (speedup 1.0000x reference)
import jax
import jax.numpy as jnp
from jax import lax
from jax.experimental import pallas as pl
from jax.experimental.pallas import tpu as pltpu

_D = 2


def _se_dual_manual_body(x_hbm, w1_ref, w2_ref, o_hbm,
                         ibufs, obufs, in_sems, out_sems, *, n_half, inv_hw):
    Nh = n_half

    def start_in(i, slot):
        pltpu.make_async_copy(x_hbm.at[i], ibufs.at[slot, 0], in_sems.at[slot, 0]).start()
        pltpu.make_async_copy(x_hbm.at[i + Nh], ibufs.at[slot, 1], in_sems.at[slot, 1]).start()

    def wait_in(i, slot):
        pltpu.make_async_copy(x_hbm.at[i], ibufs.at[slot, 0], in_sems.at[slot, 0]).wait()
        pltpu.make_async_copy(x_hbm.at[i + Nh], ibufs.at[slot, 1], in_sems.at[slot, 1]).wait()

    def start_out(i, slot):
        pltpu.make_async_copy(obufs.at[slot, 0], o_hbm.at[i], out_sems.at[slot, 0]).start()
        pltpu.make_async_copy(obufs.at[slot, 1], o_hbm.at[i + Nh], out_sems.at[slot, 1]).start()

    def wait_out(i, slot):
        pltpu.make_async_copy(obufs.at[slot, 0], o_hbm.at[i], out_sems.at[slot, 0]).wait()
        pltpu.make_async_copy(obufs.at[slot, 1], o_hbm.at[i + Nh], out_sems.at[slot, 1]).wait()

    for i in range(min(_D, Nh)):
        start_in(i, i)

    def step(i, _):
        slot = lax.rem(i, _D)
        wait_in(i, slot)

        @pl.when(i >= _D)
        def _():
            wait_out(i - _D, slot)

        xa = ibufs[slot, 0]
        xb = ibufs[slot, 1]
        pa = jnp.sum(xa, axis=1, keepdims=True)
        pb = jnp.sum(xb, axis=1, keepdims=True)
        p = jnp.concatenate([pa, pb], axis=1) * jnp.float32(inv_hw)
        s1 = jnp.maximum(
            lax.dot_general(w1_ref[...], p, (((1,), (0,)), ((), ())),
                            preferred_element_type=jnp.float32),
            0.0,
        )
        z = lax.dot_general(w2_ref[...], s1, (((1,), (0,)), ((), ())),
                            preferred_element_type=jnp.float32)
        gate = jax.nn.sigmoid(z).astype(xa.dtype)
        obufs[slot, 0] = xa * gate[:, 0:1]
        obufs[slot, 1] = xb * gate[:, 1:2]

        start_out(i, slot)

        @pl.when(i + _D < Nh)
        def _():
            start_in(i + _D, slot)

        return ()

    lax.fori_loop(0, Nh, step, (), unroll=False)

    for i in range(max(Nh - _D, 0), Nh):
        wait_out(i, lax.rem(i, _D))


def kernel(x_nchw, w1, w2):
    import functools
    N, C, H, W = x_nchw.shape
    HW = H * W
    Nh = N // 2

    x_flat = x_nchw.reshape(N, C, HW)

    out_flat = pl.pallas_call(
        functools.partial(_se_dual_manual_body, n_half=Nh, inv_hw=1.0 / HW),
        out_shape=jax.ShapeDtypeStruct((N, C, HW), x_nchw.dtype),
        in_specs=[
            pl.BlockSpec(memory_space=pl.ANY),
            pl.BlockSpec(memory_space=pltpu.VMEM),
            pl.BlockSpec(memory_space=pltpu.VMEM),
        ],
        out_specs=pl.BlockSpec(memory_space=pl.ANY),
        scratch_shapes=[
            pltpu.VMEM((_D, 2, C, HW), x_nchw.dtype),
            pltpu.VMEM((_D, 2, C, HW), x_nchw.dtype),
            pltpu.SemaphoreType.DMA((_D, 2)),
            pltpu.SemaphoreType.DMA((_D, 2)),
        ],
        compiler_params=pltpu.CompilerParams(
            vmem_limit_bytes=64 * 1024 * 1024,
        ),
    )(x_flat, w1, w2)

    return out_flat.reshape(N, C, H, W)


# speedup vs baseline: 1.0006x; 1.0006x over previous
import functools

import jax
import jax.numpy as jnp
from jax import lax
from jax.experimental import pallas as pl
from jax.experimental.pallas import tpu as pltpu


def _se_dual_body(xa_ref, xb_ref, w1_ref, w2_ref, o_ref, *, inv_hw):
    xa = xa_ref[0, 0]
    xb = xb_ref[0, 0]
    pa = jnp.sum(xa, axis=1, keepdims=True)
    pb = jnp.sum(xb, axis=1, keepdims=True)
    p = jnp.concatenate([pa, pb], axis=1) * jnp.float32(inv_hw)
    s1 = jnp.maximum(
        lax.dot_general(w1_ref[...], p, (((1,), (0,)), ((), ())),
                        preferred_element_type=jnp.float32),
        0.0,
    )
    z = lax.dot_general(w2_ref[...], s1, (((1,), (0,)), ((), ())),
                        preferred_element_type=jnp.float32)
    gate = jax.nn.sigmoid(z).astype(xa.dtype)
    o_ref[0, 0] = xa * gate[:, 0:1]
    o_ref[1, 0] = xb * gate[:, 1:2]


def kernel(x_nchw, w1, w2):
    N, C, H, W = x_nchw.shape
    HW = H * W
    Cr = w1.shape[0]
    Nh = N // 2

    x2 = x_nchw.reshape(2, Nh, C, HW)

    out2 = pl.pallas_call(
        functools.partial(_se_dual_body, inv_hw=1.0 / HW),
        out_shape=jax.ShapeDtypeStruct((2, Nh, C, HW), x_nchw.dtype),
        grid=(Nh,),
        in_specs=[
            pl.BlockSpec((1, 1, C, HW), lambda n: (0, n, 0, 0)),
            pl.BlockSpec((1, 1, C, HW), lambda n: (1, n, 0, 0)),
            pl.BlockSpec((Cr, C), lambda n: (0, 0)),
            pl.BlockSpec((C, Cr), lambda n: (0, 0)),
        ],
        out_specs=pl.BlockSpec((2, 1, C, HW), lambda n: (0, n, 0, 0)),
        compiler_params=pltpu.CompilerParams(
            dimension_semantics=("parallel",),
            vmem_limit_bytes=64 * 1024 * 1024,
        ),
    )(x2, x2, w1, w2)

    return out2.reshape(N, C, H, W)
